# baseline (speedup 1.0000x reference)
import functools

import jax
import jax.numpy as jnp
from jax import lax
from jax.experimental import pallas as pl
from jax.experimental.pallas import tpu as pltpu
from jax.experimental.pallas import tpu_sc as plsc

B = 16384
D = 32
HIDDEN = 64
NC = 2
NS = 16
NW = NC * NS
BPW = B // NW
CHUNK = 128
NCH = BPW // CHUNK

MB = 2048


def _sc_gather(uidx2d, didx2d, user_table, deal_table):
    mesh = plsc.VectorSubcoreMesh(core_axis_name="c", subcore_axis_name="s")

    @functools.partial(
        pl.kernel,
        mesh=mesh,
        out_type=[
            jax.ShapeDtypeStruct((B, D), jnp.float32),
            jax.ShapeDtypeStruct((B, D), jnp.float32),
        ],
        scratch_types=[
            pltpu.VMEM((NCH, CHUNK), jnp.int32),
            pltpu.VMEM((NCH, CHUNK), jnp.int32),
            pltpu.VMEM((BPW, D), jnp.float32),
            pltpu.VMEM((BPW, D), jnp.float32),
            pltpu.SemaphoreType.DMA,
            pltpu.SemaphoreType.DMA,
        ],
        compiler_params=pltpu.CompilerParams(use_tc_tiling_on_sc=False),
    )
    def gather_kernel(uidx_hbm, didx_hbm, utab_hbm, dtab_hbm,
                      u_out, d_out,
                      uidx_v, didx_v, urows_v, drows_v, usem, dsem):
        wid = lax.axis_index("s") * NC + lax.axis_index("c")
        pltpu.sync_copy(uidx_hbm.at[pl.ds(wid * NCH, NCH)], uidx_v)
        pltpu.sync_copy(didx_hbm.at[pl.ds(wid * NCH, NCH)], didx_v)
        copies = []
        for j in range(NCH):
            copies.append(pltpu.async_copy(
                utab_hbm.at[uidx_v.at[j]],
                urows_v.at[pl.ds(j * CHUNK, CHUNK)], usem))
            copies.append(pltpu.async_copy(
                dtab_hbm.at[didx_v.at[j]],
                drows_v.at[pl.ds(j * CHUNK, CHUNK)], dsem))
        for c in copies:
            c.wait()
        base = wid * BPW
        pltpu.sync_copy(urows_v, u_out.at[pl.ds(base, BPW)])
        pltpu.sync_copy(drows_v, d_out.at[pl.ds(base, BPW)])

    return gather_kernel(uidx2d, didx2d, user_table, deal_table)


def _mlp_body(u_ref, d_ref, img_ref, w1u_ref, w1d_ref, w1i_ref,
              b1_ref, w2t_ref, b2_ref, out_ref):
    acc = (jnp.dot(u_ref[...], w1u_ref[...], preferred_element_type=jnp.float32)
           + jnp.dot(d_ref[...], w1d_ref[...], preferred_element_type=jnp.float32)
           + jnp.dot(img_ref[...], w1i_ref[...], preferred_element_type=jnp.float32))
    h = jnp.maximum(acc + b1_ref[...], 0.0)
    out_ref[...] = jnp.sum(h * w2t_ref[...], axis=1) + b2_ref[0]


def kernel(user_idx, deal_idx, image_vec, user_table, deal_table, W1, b1, W2, b2):
    uidx2d = user_idx.astype(jnp.int32).reshape(B // CHUNK, CHUNK)
    didx2d = deal_idx.astype(jnp.int32).reshape(B // CHUNK, CHUNK)
    u, dd = _sc_gather(uidx2d, didx2d, user_table, deal_table)

    w1u, w1d, w1i = W1[:D], W1[D:2 * D], W1[2 * D:]
    b1r = b1.reshape(1, HIDDEN)
    w2t = W2.reshape(1, HIDDEN)

    score = pl.pallas_call(
        _mlp_body,
        grid=(B // MB,),
        in_specs=[
            pl.BlockSpec((MB, D), lambda i: (i, 0)),
            pl.BlockSpec((MB, D), lambda i: (i, 0)),
            pl.BlockSpec((MB, D), lambda i: (i, 0)),
            pl.BlockSpec((D, HIDDEN), lambda i: (0, 0)),
            pl.BlockSpec((D, HIDDEN), lambda i: (0, 0)),
            pl.BlockSpec((D, HIDDEN), lambda i: (0, 0)),
            pl.BlockSpec((1, HIDDEN), lambda i: (0, 0)),
            pl.BlockSpec((1, HIDDEN), lambda i: (0, 0)),
            pl.BlockSpec(memory_space=pltpu.SMEM),
        ],
        out_specs=pl.BlockSpec((MB,), lambda i: (i,)),
        out_shape=jax.ShapeDtypeStruct((B,), jnp.float32),
    )(u, dd, image_vec, w1u, w1d, w1i, b1r, w2t, b2)
    return score
